# Initial kernel scaffold; baseline (speedup 1.0000x reference)
import jax
import jax.numpy as jnp
from jax.experimental import pallas as pl


def kernel(x, edge_index, W1, a_src1, a_dst1, b1, W2, a_src2, a_dst2, b2, W3, a_src3, a_dst3, b3):
    raise NotImplementedError("write your pallas kernel here")


# speedup vs baseline: 15.4443x; 15.4443x over previous
import functools

import jax
import jax.numpy as jnp
from jax import lax
from jax.experimental import pallas as pl
from jax.experimental.pallas import tpu as pltpu
from jax.experimental.pallas import tpu_sc as plsc

N = 10000
E = 320000
EALL = E + N
OUT = 64
HEADS = 8
HC = 32

NTILE = 16
CHUNK = 64
EPT = -(-EALL // (NTILE * CHUNK)) * CHUNK
EPAD = EPT * NTILE
NPT = 624
NPT_LAST = N - NPT * (NTILE - 1)


def _tc_body(act_ref, w_ref, aws_ref, awd_ref,
             h_ref, as_ref, ad_ref, mxs_ref, mxd_ref):
    i = pl.program_id(0)
    h = jnp.dot(act_ref[...], w_ref[...], preferred_element_type=jnp.float32)
    h_ref[...] = h
    a_s = jnp.dot(h, aws_ref[...], preferred_element_type=jnp.float32)
    a_d = jnp.dot(h, awd_ref[...], preferred_element_type=jnp.float32)
    as_ref[...] = a_s
    ad_ref[...] = a_d
    ms = jnp.broadcast_to(jnp.max(a_s, axis=0, keepdims=True), (8, 8))
    md = jnp.broadcast_to(jnp.max(a_d, axis=0, keepdims=True), (8, 8))

    @pl.when(i == 0)
    def _():
        mxs_ref[...] = ms
        mxd_ref[...] = md

    @pl.when(i > 0)
    def _():
        mxs_ref[...] = jnp.maximum(mxs_ref[...], ms)
        mxd_ref[...] = jnp.maximum(mxd_ref[...], md)


def _tc_stage(act, w, aws, awd):
    d, f = w.shape
    bn = 1000
    grid = (N // bn,)
    return pl.pallas_call(
        _tc_body,
        grid=grid,
        in_specs=[
            pl.BlockSpec((bn, d), lambda i: (i, 0)),
            pl.BlockSpec((d, f), lambda i: (0, 0)),
            pl.BlockSpec((f, 8), lambda i: (0, 0)),
            pl.BlockSpec((f, 8), lambda i: (0, 0)),
        ],
        out_specs=[
            pl.BlockSpec((bn, f), lambda i: (i, 0)),
            pl.BlockSpec((bn, 8), lambda i: (i, 0)),
            pl.BlockSpec((bn, 8), lambda i: (i, 0)),
            pl.BlockSpec((8, 8), lambda i: (0, 0)),
            pl.BlockSpec((8, 8), lambda i: (0, 0)),
        ],
        out_shape=[
            jax.ShapeDtypeStruct((N, f), jnp.float32),
            jax.ShapeDtypeStruct((N, 8), jnp.float32),
            jax.ShapeDtypeStruct((N, 8), jnp.float32),
            jax.ShapeDtypeStruct((8, 8), jnp.float32),
            jax.ShapeDtypeStruct((8, 8), jnp.float32),
        ],
    )(act, w, aws, awd)


def _make_sc_layer(hp, ch, elu, attn_c0_only):
    epv = 16 // hp
    nv = CHUNK // epv
    vph = (ch // hp) // 16
    nch = EPT // CHUNK
    nsc = max(1, (CHUNK * hp) // 128)
    scw = (CHUNK * hp) // nsc
    mesh = plsc.VectorSubcoreMesh(core_axis_name="c", subcore_axis_name="s")

    @functools.partial(
        pl.kernel,
        out_type=[
            jax.ShapeDtypeStruct((2 * N, ch), jnp.float32),
            jax.ShapeDtypeStruct((2 * EPAD * hp,), jnp.float32),
        ],
        mesh=mesh,
        scratch_types=[
            pltpu.VMEM_SHARED((N, ch), jnp.float32),
            pltpu.VMEM_SHARED((N * hp,), jnp.float32),
            pltpu.VMEM_SHARED((2 * N * hp,), jnp.float32),
            pltpu.VMEM((CHUNK,), jnp.int32),
            pltpu.VMEM((CHUNK,), jnp.int32),
            pltpu.VMEM((CHUNK,), jnp.int32),
            pltpu.VMEM((nsc, scw), jnp.int32),
            pltpu.VMEM((nsc, scw), jnp.int32),
            pltpu.VMEM((nsc, scw), jnp.int32),
            pltpu.VMEM((CHUNK, ch), jnp.float32),
            pltpu.VMEM((CHUNK, ch), jnp.float32),
            pltpu.VMEM((CHUNK * hp,), jnp.float32),
            pltpu.VMEM((CHUNK * hp,), jnp.float32),
            pltpu.VMEM((CHUNK * hp,), jnp.float32),
            pltpu.VMEM((NPT_LAST * hp,), jnp.float32),
            pltpu.VMEM((16,), jnp.float32),
            pltpu.VMEM((ch,), jnp.float32),
            pltpu.VMEM((CHUNK, ch), jnp.float32),
            pltpu.SemaphoreType.DMA,
        ],
        compiler_params=pltpu.CompilerParams(needs_layout_passes=False,
                                             use_tc_tiling_on_sc=False),
    )
    def sc_fn(h_hbm, aw_hbm, shift_hbm, bias_hbm, ei_hbm, out_hbm, attn_hbm,
              u_sh, den_sh, tbl, src_v, dst_v, idx_v, si_v, ai_v, di_v,
              hrows, mrows, exv, gsv, dnv, zb1, shv, bv, orows, sem):
        iota = lax.iota(jnp.int32, 16)
        c = lax.axis_index("c")
        t = lax.axis_index("s")
        cN = c * N

        @pl.when(t == 0)
        def _():
            pltpu.sync_copy(aw_hbm.at[pl.ds(c * (2 * N * hp), 2 * N * hp)],
                            tbl)
        pltpu.sync_copy(shift_hbm.at[pl.ds(c * 16, 16)], shv)
        pltpu.sync_copy(bias_hbm.at[pl.ds(c * ch, ch)], bv)

        zbuf = mrows
        def _zero_2d(v, _):
            zbuf[v // (ch // 16), pl.ds((v % (ch // 16)) * 16, 16)] = (
                jnp.zeros((16,), jnp.float32))
            return 0
        lax.fori_loop(0, CHUNK * (ch // 16), _zero_2d, 0)
        def _zero_1d(k, _):
            zb1[pl.ds(k * 16, 16)] = jnp.zeros((16,), jnp.float32)
            return 0
        lax.fori_loop(0, NPT_LAST * hp // 16, _zero_1d, 0)

        my_n0 = t * NPT
        nfull = NPT // CHUNK
        def _zero_u(k, _):
            pltpu.sync_copy(zbuf, u_sh.at[pl.ds(my_n0 + k * CHUNK, CHUNK)])
            return 0
        lax.fori_loop(0, nfull, _zero_u, 0)

        @pl.when(t == NTILE - 1)
        def _():
            pltpu.sync_copy(zbuf,
                            u_sh.at[pl.ds(my_n0 + nfull * CHUNK, CHUNK)])
            pltpu.sync_copy(zb1, den_sh.at[pl.ds(my_n0 * hp,
                                                 NPT_LAST * hp)])

        @pl.when(t < NTILE - 1)
        def _():
            pltpu.sync_copy(zbuf.at[pl.ds(0, NPT - nfull * CHUNK)],
                            u_sh.at[pl.ds(my_n0 + nfull * CHUNK,
                                          NPT - nfull * CHUNK)])
            pltpu.sync_copy(zb1.at[pl.ds(0, NPT * hp)],
                            den_sh.at[pl.ds(my_n0 * hp, NPT * hp)])
        plsc.subcore_barrier()

        shift_vec = shv[...]
        ebase = t * EPT

        def _chunk_a(ci, _):
            base = ebase + ci * CHUNK
            pltpu.sync_copy(ei_hbm.at[pl.ds(base, CHUNK)], src_v)
            pltpu.sync_copy(ei_hbm.at[pl.ds(EPAD + base, CHUNK)], dst_v)

            def _mkidx(k, _):
                idx_v[pl.ds(k * 16, 16)] = src_v[pl.ds(k * 16, 16)] + cN
                return 0
            lax.fori_loop(0, CHUNK // 16, _mkidx, 0)
            cp = pltpu.async_copy(h_hbm.at[idx_v], hrows, sem)

            def _mkai(v, _):
                e0 = v * epv
                hcol = iota % hp
                sr = plsc.load_gather(src_v, [iota // hp + e0])
                dr = plsc.load_gather(dst_v, [iota // hp + e0])
                j = v // (scw // 16)
                o = (v % (scw // 16)) * 16
                si_v[j, pl.ds(o, 16)] = sr * hp + hcol
                di = dr * hp + hcol
                di_v[j, pl.ds(o, 16)] = di
                ai_v[j, pl.ds(o, 16)] = di + N * hp
                return 0
            lax.fori_loop(0, nv, _mkai, 0)
            for j in range(nsc):
                pltpu.sync_copy(tbl.at[si_v.at[j]],
                                gsv.at[pl.ds(j * scw, scw)])
                pltpu.sync_copy(tbl.at[ai_v.at[j]],
                                dnv.at[pl.ds(j * scw, scw)])

            def _exv(v, _):
                e0 = v * epv
                al = gsv[pl.ds(v * 16, 16)] + dnv[pl.ds(v * 16, 16)]
                al = jnp.maximum(al, 0.0) + 0.2 * jnp.minimum(al, 0.0)
                ex = jnp.exp(al - shift_vec)
                gid = base + e0 + iota // hp
                ex = jnp.where(gid < EALL, ex, 0.0)
                exv[pl.ds(v * 16, 16)] = ex
                return 0
            lax.fori_loop(0, nv, _exv, 0)
            cp.wait()

            def _msg(e, _):
                for v in range(ch // 16):
                    hv = hrows[e, pl.ds(v * 16, 16)]
                    bc = plsc.load_gather(
                        exv, [jnp.full((16,), e * hp + v // vph, jnp.int32)])
                    mrows[e, pl.ds(v * 16, 16)] = hv * bc
                return 0
            lax.fori_loop(0, CHUNK, _msg, 0)

            pltpu.sync_copy(mrows, u_sh.at[dst_v], add=True)
            for j in range(nsc):
                pltpu.sync_copy(exv.at[pl.ds(j * scw, scw)],
                                den_sh.at[di_v.at[j]], add=True)
            pltpu.sync_copy(exv,
                            attn_hbm.at[pl.ds((c * EPAD + base) * hp,
                                              CHUNK * hp)])
            return 0
        lax.fori_loop(0, nch, _chunk_a, 0)

        plsc.subcore_barrier()

        bias_vs = [bv[pl.ds(v * 16, 16)] for v in range(ch // 16)]

        def _node_block(r0, nrow):
            pltpu.sync_copy(u_sh.at[pl.ds(r0, nrow)], mrows.at[pl.ds(0, nrow)])
            pltpu.sync_copy(den_sh.at[pl.ds(r0 * hp, nrow * hp)],
                            exv.at[pl.ds(0, nrow * hp)])

            def _row(r, _):
                for v in range(ch // 16):
                    uv = mrows[r, pl.ds(v * 16, 16)]
                    db = plsc.load_gather(
                        exv, [jnp.full((16,), r * hp + v // vph, jnp.int32)])
                    ov = uv / (db + 1e-16) + bias_vs[v]
                    if elu:
                        ov = jnp.where(ov > 0.0, ov,
                                       jnp.exp(jnp.minimum(ov, 0.0)) - 1.0)
                    orows[r, pl.ds(v * 16, 16)] = ov
                return 0
            lax.fori_loop(0, nrow, _row, 0)
            pltpu.sync_copy(orows.at[pl.ds(0, nrow)],
                            out_hbm.at[pl.ds(cN + r0, nrow)])

        nb = NPT // CHUNK
        def _b1(k, _):
            _node_block(t * NPT + k * CHUNK, CHUNK)
            return 0
        lax.fori_loop(0, nb, _b1, 0)

        @pl.when(t == NTILE - 1)
        def _():
            _node_block(t * NPT + nb * CHUNK, CHUNK)

        @pl.when(t < NTILE - 1)
        def _():
            _node_block(t * NPT + nb * CHUNK, NPT - nb * CHUNK)

        def _chunk_b(ci, _):
            base = ebase + ci * CHUNK
            pltpu.sync_copy(ei_hbm.at[pl.ds(EPAD + base, CHUNK)], dst_v)

            def _mkdi(v, _):
                dr = plsc.load_gather(dst_v, [iota // hp + v * epv])
                di_v[v // (scw // 16), pl.ds((v % (scw // 16)) * 16, 16)] = (
                    dr * hp + iota % hp)
                return 0
            lax.fori_loop(0, nv, _mkdi, 0)
            for j in range(nsc):
                pltpu.sync_copy(den_sh.at[di_v.at[j]],
                                dnv.at[pl.ds(j * scw, scw)])
            pltpu.sync_copy(
                attn_hbm.at[pl.ds((c * EPAD + base) * hp, CHUNK * hp)], exv)

            def _att(v, _):
                ex = exv[pl.ds(v * 16, 16)]
                db = dnv[pl.ds(v * 16, 16)]
                exv[pl.ds(v * 16, 16)] = ex / (db + 1e-16)
                return 0
            lax.fori_loop(0, nv, _att, 0)
            pltpu.sync_copy(exv,
                            attn_hbm.at[pl.ds((c * EPAD + base) * hp,
                                              CHUNK * hp)])
            return 0

        if attn_c0_only:
            @pl.when(c == 0)
            def _():
                lax.fori_loop(0, nch, _chunk_b, 0)
        else:
            lax.fori_loop(0, nch, _chunk_b, 0)

    return sc_fn


_sc_layer12 = _make_sc_layer(4, 128, True, False)
_sc_layer3 = _make_sc_layer(1, 32, False, True)


def _lrelu(x):
    return jnp.maximum(x, 0.0) + 0.2 * jnp.minimum(x, 0.0)


def _gat12(act, w, att_s, att_d, bias, ei_flat):
    kr = jnp.kron(jnp.eye(HEADS, dtype=jnp.float32),
                  jnp.ones((HC, 1), jnp.float32))
    aws = kr * att_s.reshape(-1, 1)
    awd = kr * att_d.reshape(-1, 1)
    h, asn, adn, mxs, mxd = _tc_stage(act, w, aws, awd)
    s = _lrelu(mxs[0] + mxd[0])
    shift_cat = jnp.concatenate(
        [jnp.tile(s[0:4], 4), jnp.tile(s[4:8], 4)], axis=0)
    aw_flat = jnp.concatenate(
        [asn[:, 0:4], adn[:, 0:4], asn[:, 4:8], adn[:, 4:8]],
        axis=0).reshape(-1)
    h_cat = jnp.concatenate([h[:, :128], h[:, 128:]], axis=0)
    out_cat, attn_flat = _sc_layer12(h_cat, aw_flat, shift_cat, bias, ei_flat)
    out = jnp.concatenate([out_cat[:N], out_cat[N:]], axis=1)
    a = attn_flat.reshape(2, EPAD, 4)
    attn = jnp.concatenate([a[0, :EALL], a[1, :EALL]], axis=1)
    return out, attn


def _gat3(act, w, att_s, att_d, bias, ei_flat):
    aws = jnp.zeros((OUT, 8), jnp.float32).at[:, 0].set(att_s.reshape(-1))
    awd = jnp.zeros((OUT, 8), jnp.float32).at[:, 0].set(att_d.reshape(-1))
    h, asn, adn, mxs, mxd = _tc_stage(act, w, aws, awd)
    s = _lrelu(mxs[0, 0] + mxd[0, 0])
    shift_cat = jnp.tile(s.reshape(1), 32)
    aw_flat = jnp.concatenate(
        [asn[:, 0], adn[:, 0], asn[:, 0], adn[:, 0]], axis=0)
    h_cat = jnp.concatenate([h[:, :32], h[:, 32:]], axis=0)
    bias_cat = jnp.concatenate([bias[:32], bias[32:]], axis=0)
    out_cat, attn_flat = _sc_layer3(h_cat, aw_flat, shift_cat, bias_cat,
                                    ei_flat)
    out = jnp.concatenate([out_cat[:N], out_cat[N:]], axis=1)
    attn = attn_flat.reshape(2, EPAD, 1)[0, :EALL]
    return out, attn


def kernel(x, edge_index, W1, a_src1, a_dst1, b1, W2, a_src2, a_dst2, b2,
           W3, a_src3, a_dst3, b3):
    loops = jnp.arange(N, dtype=edge_index.dtype)
    ei = jnp.concatenate([edge_index, jnp.stack([loops, loops], axis=0)],
                         axis=1)
    pad = jnp.zeros((2, EPAD - EALL), edge_index.dtype)
    ei_flat = jnp.concatenate([ei, pad], axis=1).reshape(-1)

    h1, attn1 = _gat12(x, W1, a_src1, a_dst1, b1, ei_flat)
    h2, attn2 = _gat12(h1, W2, a_src2, a_dst2, b2, ei_flat)
    out, attn3 = _gat3(h2, W3, a_src3, a_dst3, b3, ei_flat)
    return (out, attn1, attn2, attn3)


# speedup vs baseline: 19.5460x; 1.2656x over previous
import functools

import jax
import jax.numpy as jnp
from jax import lax
from jax.experimental import pallas as pl
from jax.experimental.pallas import tpu as pltpu
from jax.experimental.pallas import tpu_sc as plsc

N = 10000
E = 320000
EALL = E + N
OUT = 64
HEADS = 8
HC = 32

NTILE = 16
CHUNK = 96
EPT = -(-EALL // (NTILE * CHUNK)) * CHUNK
EPAD = EPT * NTILE
NPT = 624
NPT_LAST = N - NPT * (NTILE - 1)


def _tc_body(act_ref, w_ref, aws_ref, awd_ref,
             h_ref, as_ref, ad_ref, mxs_ref, mxd_ref):
    i = pl.program_id(0)
    h = jnp.dot(act_ref[...], w_ref[...], preferred_element_type=jnp.float32)
    h_ref[...] = h
    a_s = jnp.dot(h, aws_ref[...], preferred_element_type=jnp.float32)
    a_d = jnp.dot(h, awd_ref[...], preferred_element_type=jnp.float32)
    as_ref[...] = a_s
    ad_ref[...] = a_d
    ms = jnp.broadcast_to(jnp.max(a_s, axis=0, keepdims=True), (8, 8))
    md = jnp.broadcast_to(jnp.max(a_d, axis=0, keepdims=True), (8, 8))

    @pl.when(i == 0)
    def _():
        mxs_ref[...] = ms
        mxd_ref[...] = md

    @pl.when(i > 0)
    def _():
        mxs_ref[...] = jnp.maximum(mxs_ref[...], ms)
        mxd_ref[...] = jnp.maximum(mxd_ref[...], md)


def _tc_stage(act, w, aws, awd):
    d, f = w.shape
    bn = 1000
    grid = (N // bn,)
    return pl.pallas_call(
        _tc_body,
        grid=grid,
        in_specs=[
            pl.BlockSpec((bn, d), lambda i: (i, 0)),
            pl.BlockSpec((d, f), lambda i: (0, 0)),
            pl.BlockSpec((f, 8), lambda i: (0, 0)),
            pl.BlockSpec((f, 8), lambda i: (0, 0)),
        ],
        out_specs=[
            pl.BlockSpec((bn, f), lambda i: (i, 0)),
            pl.BlockSpec((bn, 8), lambda i: (i, 0)),
            pl.BlockSpec((bn, 8), lambda i: (i, 0)),
            pl.BlockSpec((8, 8), lambda i: (0, 0)),
            pl.BlockSpec((8, 8), lambda i: (0, 0)),
        ],
        out_shape=[
            jax.ShapeDtypeStruct((N, f), jnp.float32),
            jax.ShapeDtypeStruct((N, 8), jnp.float32),
            jax.ShapeDtypeStruct((N, 8), jnp.float32),
            jax.ShapeDtypeStruct((8, 8), jnp.float32),
            jax.ShapeDtypeStruct((8, 8), jnp.float32),
        ],
    )(act, w, aws, awd)


def _make_sc_layer(hp, ch, elu, attn_c0_only):
    epv = 16 // hp
    nv = CHUNK // epv
    vph = (ch // hp) // 16
    nch = EPT // CHUNK
    nsc = max(1, (CHUNK * hp) // 128)
    scw = (CHUNK * hp) // nsc
    mesh = plsc.VectorSubcoreMesh(core_axis_name="c", subcore_axis_name="s")

    @functools.partial(
        pl.kernel,
        out_type=[
            jax.ShapeDtypeStruct((2 * N, ch), jnp.float32),
            jax.ShapeDtypeStruct((2 * EPAD * hp,), jnp.float32),
        ],
        mesh=mesh,
        scratch_types=[
            pltpu.VMEM_SHARED((N, ch), jnp.float32),
            pltpu.VMEM_SHARED((N * hp,), jnp.float32),
            pltpu.VMEM_SHARED((2 * N * hp,), jnp.float32),
            pltpu.VMEM((CHUNK,), jnp.int32),
            pltpu.VMEM((CHUNK,), jnp.int32),
            pltpu.VMEM((CHUNK,), jnp.int32),
            pltpu.VMEM((nsc, scw), jnp.int32),
            pltpu.VMEM((nsc, scw), jnp.int32),
            pltpu.VMEM((nsc, scw), jnp.int32),
            pltpu.VMEM((CHUNK, ch), jnp.float32),
            pltpu.VMEM((CHUNK * hp,), jnp.float32),
            pltpu.VMEM((CHUNK * hp,), jnp.float32),
            pltpu.VMEM((CHUNK * hp,), jnp.float32),
            pltpu.VMEM((NPT_LAST * hp,), jnp.float32),
            pltpu.VMEM((16,), jnp.float32),
            pltpu.VMEM((ch,), jnp.float32),
            pltpu.VMEM((CHUNK, ch), jnp.float32),
            pltpu.SemaphoreType.DMA,
        ],
        compiler_params=pltpu.CompilerParams(needs_layout_passes=False,
                                             use_tc_tiling_on_sc=False),
    )
    def sc_fn(h_hbm, aw_hbm, shift_hbm, bias_hbm, ei_hbm, out_hbm, attn_hbm,
              u_sh, den_sh, tbl, src_v, dst_v, idx_v, si_v, ai_v, di_v,
              hrows, exv, gsv, dnv, zb1, shv, bv, orows, sem):
        iota = lax.iota(jnp.int32, 16)
        c = lax.axis_index("c")
        t = lax.axis_index("s")
        cN = c * N

        @pl.when(t == 0)
        def _():
            pltpu.sync_copy(aw_hbm.at[pl.ds(c * (2 * N * hp), 2 * N * hp)],
                            tbl)
        pltpu.sync_copy(shift_hbm.at[pl.ds(c * 16, 16)], shv)
        pltpu.sync_copy(bias_hbm.at[pl.ds(c * ch, ch)], bv)

        zbuf = orows
        def _zero_2d(v, _):
            zbuf[v // (ch // 16), pl.ds((v % (ch // 16)) * 16, 16)] = (
                jnp.zeros((16,), jnp.float32))
            return 0
        lax.fori_loop(0, CHUNK * (ch // 16), _zero_2d, 0)
        def _zero_1d(k, _):
            zb1[pl.ds(k * 16, 16)] = jnp.zeros((16,), jnp.float32)
            return 0
        lax.fori_loop(0, NPT_LAST * hp // 16, _zero_1d, 0)

        my_n0 = t * NPT
        nfull = NPT // CHUNK
        def _zero_u(k, _):
            pltpu.sync_copy(zbuf, u_sh.at[pl.ds(my_n0 + k * CHUNK, CHUNK)])
            return 0
        lax.fori_loop(0, nfull, _zero_u, 0)

        @pl.when(t == NTILE - 1)
        def _():
            pltpu.sync_copy(zbuf.at[pl.ds(0, NPT_LAST - nfull * CHUNK)],
                            u_sh.at[pl.ds(my_n0 + nfull * CHUNK,
                                          NPT_LAST - nfull * CHUNK)])
            pltpu.sync_copy(zb1, den_sh.at[pl.ds(my_n0 * hp,
                                                 NPT_LAST * hp)])

        @pl.when(t < NTILE - 1)
        def _():
            pltpu.sync_copy(zbuf.at[pl.ds(0, NPT - nfull * CHUNK)],
                            u_sh.at[pl.ds(my_n0 + nfull * CHUNK,
                                          NPT - nfull * CHUNK)])
            pltpu.sync_copy(zb1.at[pl.ds(0, NPT * hp)],
                            den_sh.at[pl.ds(my_n0 * hp, NPT * hp)])
        plsc.subcore_barrier()

        shift_vec = shv[...]
        ebase = t * EPT

        def _chunk_a(ci, _):
            base = ebase + ci * CHUNK
            pltpu.sync_copy(ei_hbm.at[pl.ds(base, CHUNK)], src_v)
            pltpu.sync_copy(ei_hbm.at[pl.ds(EPAD + base, CHUNK)], dst_v)

            def _mkidx(k, _):
                idx_v[pl.ds(k * 16, 16)] = src_v[pl.ds(k * 16, 16)] + cN
                return 0
            lax.fori_loop(0, CHUNK // 16, _mkidx, 0)
            cp = pltpu.async_copy(h_hbm.at[idx_v], hrows, sem)

            def _mkai(v, _):
                e0 = v * epv
                hcol = iota % hp
                sr = plsc.load_gather(src_v, [iota // hp + e0])
                dr = plsc.load_gather(dst_v, [iota // hp + e0])
                j = v // (scw // 16)
                o = (v % (scw // 16)) * 16
                si_v[j, pl.ds(o, 16)] = sr * hp + hcol
                di = dr * hp + hcol
                di_v[j, pl.ds(o, 16)] = di
                ai_v[j, pl.ds(o, 16)] = di + N * hp
                return 0
            lax.fori_loop(0, nv, _mkai, 0)
            for j in range(nsc):
                pltpu.sync_copy(tbl.at[si_v.at[j]],
                                gsv.at[pl.ds(j * scw, scw)])
                pltpu.sync_copy(tbl.at[ai_v.at[j]],
                                dnv.at[pl.ds(j * scw, scw)])

            def _exv(v, _):
                e0 = v * epv
                al = gsv[pl.ds(v * 16, 16)] + dnv[pl.ds(v * 16, 16)]
                al = jnp.maximum(al, 0.0) + 0.2 * jnp.minimum(al, 0.0)
                ex = jnp.exp(al - shift_vec)
                gid = base + e0 + iota // hp
                ex = jnp.where(gid < EALL, ex, 0.0)
                exv[pl.ds(v * 16, 16)] = ex
                return 0
            lax.fori_loop(0, nv, _exv, 0)
            cp.wait()

            def _msg(e, _):
                for hd in range(hp):
                    bc = plsc.load_gather(
                        exv, [jnp.full((16,), e * hp + hd, jnp.int32)])
                    for v in range(vph):
                        k = (hd * vph + v) * 16
                        hrows[e, pl.ds(k, 16)] = hrows[e, pl.ds(k, 16)] * bc
                return 0
            lax.fori_loop(0, CHUNK, _msg, 0)

            pltpu.sync_copy(hrows, u_sh.at[dst_v], add=True)
            for j in range(nsc):
                pltpu.sync_copy(exv.at[pl.ds(j * scw, scw)],
                                den_sh.at[di_v.at[j]], add=True)
            pltpu.sync_copy(exv,
                            attn_hbm.at[pl.ds((c * EPAD + base) * hp,
                                              CHUNK * hp)])
            return 0
        lax.fori_loop(0, nch, _chunk_a, 0)

        plsc.subcore_barrier()

        bias_vs = [bv[pl.ds(v * 16, 16)] for v in range(ch // 16)]

        def _node_block(r0, nrow):
            pltpu.sync_copy(u_sh.at[pl.ds(r0, nrow)], orows.at[pl.ds(0, nrow)])
            pltpu.sync_copy(den_sh.at[pl.ds(r0 * hp, nrow * hp)],
                            exv.at[pl.ds(0, nrow * hp)])

            def _row(r, _):
                for v in range(ch // 16):
                    uv = orows[r, pl.ds(v * 16, 16)]
                    db = plsc.load_gather(
                        exv, [jnp.full((16,), r * hp + v // vph, jnp.int32)])
                    ov = uv / (db + 1e-16) + bias_vs[v]
                    if elu:
                        ov = jnp.where(ov > 0.0, ov,
                                       jnp.exp(jnp.minimum(ov, 0.0)) - 1.0)
                    orows[r, pl.ds(v * 16, 16)] = ov
                return 0
            lax.fori_loop(0, nrow, _row, 0)
            pltpu.sync_copy(orows.at[pl.ds(0, nrow)],
                            out_hbm.at[pl.ds(cN + r0, nrow)])

        nb = NPT // CHUNK
        def _b1(k, _):
            _node_block(t * NPT + k * CHUNK, CHUNK)
            return 0
        lax.fori_loop(0, nb, _b1, 0)

        @pl.when(t == NTILE - 1)
        def _():
            _node_block(t * NPT + nb * CHUNK, NPT_LAST - nb * CHUNK)

        @pl.when(t < NTILE - 1)
        def _():
            _node_block(t * NPT + nb * CHUNK, NPT - nb * CHUNK)

        def _chunk_b(ci, _):
            base = ebase + ci * CHUNK
            pltpu.sync_copy(ei_hbm.at[pl.ds(EPAD + base, CHUNK)], dst_v)

            def _mkdi(v, _):
                dr = plsc.load_gather(dst_v, [iota // hp + v * epv])
                di_v[v // (scw // 16), pl.ds((v % (scw // 16)) * 16, 16)] = (
                    dr * hp + iota % hp)
                return 0
            lax.fori_loop(0, nv, _mkdi, 0)
            for j in range(nsc):
                pltpu.sync_copy(den_sh.at[di_v.at[j]],
                                dnv.at[pl.ds(j * scw, scw)])
            pltpu.sync_copy(
                attn_hbm.at[pl.ds((c * EPAD + base) * hp, CHUNK * hp)], exv)

            def _att(v, _):
                ex = exv[pl.ds(v * 16, 16)]
                db = dnv[pl.ds(v * 16, 16)]
                exv[pl.ds(v * 16, 16)] = ex / (db + 1e-16)
                return 0
            lax.fori_loop(0, nv, _att, 0)
            pltpu.sync_copy(exv,
                            attn_hbm.at[pl.ds((c * EPAD + base) * hp,
                                              CHUNK * hp)])
            return 0

        if attn_c0_only:
            @pl.when(c == 0)
            def _():
                lax.fori_loop(0, nch, _chunk_b, 0)
        else:
            lax.fori_loop(0, nch, _chunk_b, 0)

    return sc_fn


_sc_layer12 = _make_sc_layer(4, 128, True, False)
_sc_layer3 = _make_sc_layer(1, 32, False, True)


def _lrelu(x):
    return jnp.maximum(x, 0.0) + 0.2 * jnp.minimum(x, 0.0)


def _gat12(act, w, att_s, att_d, bias, ei_flat):
    kr = jnp.kron(jnp.eye(HEADS, dtype=jnp.float32),
                  jnp.ones((HC, 1), jnp.float32))
    aws = kr * att_s.reshape(-1, 1)
    awd = kr * att_d.reshape(-1, 1)
    h, asn, adn, mxs, mxd = _tc_stage(act, w, aws, awd)
    s = _lrelu(mxs[0] + mxd[0])
    shift_cat = jnp.concatenate(
        [jnp.tile(s[0:4], 4), jnp.tile(s[4:8], 4)], axis=0)
    aw_flat = jnp.concatenate(
        [asn[:, 0:4], adn[:, 0:4], asn[:, 4:8], adn[:, 4:8]],
        axis=0).reshape(-1)
    h_cat = jnp.concatenate([h[:, :128], h[:, 128:]], axis=0)
    out_cat, attn_flat = _sc_layer12(h_cat, aw_flat, shift_cat, bias, ei_flat)
    out = jnp.concatenate([out_cat[:N], out_cat[N:]], axis=1)
    a = attn_flat.reshape(2, EPAD, 4)
    attn = jnp.concatenate([a[0, :EALL], a[1, :EALL]], axis=1)
    return out, attn


def _gat3(act, w, att_s, att_d, bias, ei_flat):
    aws = jnp.zeros((OUT, 8), jnp.float32).at[:, 0].set(att_s.reshape(-1))
    awd = jnp.zeros((OUT, 8), jnp.float32).at[:, 0].set(att_d.reshape(-1))
    h, asn, adn, mxs, mxd = _tc_stage(act, w, aws, awd)
    s = _lrelu(mxs[0, 0] + mxd[0, 0])
    shift_cat = jnp.tile(s.reshape(1), 32)
    aw_flat = jnp.concatenate(
        [asn[:, 0], adn[:, 0], asn[:, 0], adn[:, 0]], axis=0)
    h_cat = jnp.concatenate([h[:, :32], h[:, 32:]], axis=0)
    bias_cat = jnp.concatenate([bias[:32], bias[32:]], axis=0)
    out_cat, attn_flat = _sc_layer3(h_cat, aw_flat, shift_cat, bias_cat,
                                    ei_flat)
    out = jnp.concatenate([out_cat[:N], out_cat[N:]], axis=1)
    attn = attn_flat.reshape(2, EPAD, 1)[0, :EALL]
    return out, attn


def kernel(x, edge_index, W1, a_src1, a_dst1, b1, W2, a_src2, a_dst2, b2,
           W3, a_src3, a_dst3, b3):
    loops = jnp.arange(N, dtype=edge_index.dtype)
    ei = jnp.concatenate([edge_index, jnp.stack([loops, loops], axis=0)],
                         axis=1)
    pad = jnp.zeros((2, EPAD - EALL), edge_index.dtype)
    ei_flat = jnp.concatenate([ei, pad], axis=1).reshape(-1)

    h1, attn1 = _gat12(x, W1, a_src1, a_dst1, b1, ei_flat)
    h2, attn2 = _gat12(h1, W2, a_src2, a_dst2, b2, ei_flat)
    out, attn3 = _gat3(h2, W3, a_src3, a_dst3, b3, ei_flat)
    return (out, attn1, attn2, attn3)
